# baseline (speedup 1.0000x reference)
import functools

import jax
import jax.numpy as jnp
from jax import lax
from jax.experimental import pallas as pl
from jax.experimental.pallas import tpu as pltpu
from jax.experimental.pallas import tpu_sc as plsc

_NC = 2
_NS = 16
_NW = _NC * _NS

_L = 16
_CH = 4096


@functools.partial(jax.jit, static_argnums=(2,))
def _sc_row_gather(tab, idx, rows_per_w):
    r_total, v = tab.shape
    f_total, b = idx.shape
    d = r_total // f_total
    n_ch = b // _CH
    mesh = plsc.VectorSubcoreMesh(core_axis_name="c", subcore_axis_name="s")

    @functools.partial(
        pl.kernel,
        out_type=jax.ShapeDtypeStruct((r_total, b), jnp.float32),
        mesh=mesh,
        scratch_types=[
            pltpu.VMEM((v,), jnp.float32),
            pltpu.VMEM((b,), jnp.int32),
            pltpu.VMEM((2, _CH), jnp.float32),
            pltpu.SemaphoreType.DMA,
        ],
        compiler_params=pltpu.CompilerParams(needs_layout_passes=False),
    )
    def k(tab_hbm, idx_hbm, out_hbm, row_v, idx_v, out_v, sem_o):
        wid = lax.axis_index("s") * _NC + lax.axis_index("c")
        row0 = wid * rows_per_w

        def drain_out():
            pltpu.make_async_copy(
                out_hbm.at[row0, pl.ds(0, _CH)], out_v.at[0], sem_o
            ).wait()

        @pl.loop(0, rows_per_w, init_carry=jnp.int32(-1))
        def _row(kk, prev_f):
            r = row0 + kk
            f = r // d

            @pl.when(f != prev_f)
            def _():
                pltpu.sync_copy(idx_hbm.at[f], idx_v)

            pltpu.sync_copy(tab_hbm.at[r], row_v)

            for c in range(n_ch):
                @pl.when(kk * n_ch + c >= 2)
                def _():
                    drain_out()

                @plsc.parallel_loop(0, _CH, step=_L, unroll=8)
                def _vec(i):
                    iv = idx_v[pl.ds(c * _CH + i, _L)]
                    out_v[c % 2, pl.ds(i, _L)] = plsc.load_gather(row_v, [iv])

                pltpu.async_copy(
                    out_v.at[c % 2], out_hbm.at[r, pl.ds(c * _CH, _CH)], sem_o
                )
            return f

        drain_out()
        drain_out()

    return k(tab, idx)


def kernel(input, tables):
    f, v, d = tables.shape
    b = input.shape[0]
    r_total = f * d
    assert r_total % _NW == 0 and b % _CH == 0
    tab_rows = tables.transpose(0, 2, 1).reshape(r_total, v)
    idx_t = input.astype(jnp.int32).T
    out = _sc_row_gather(tab_rows, idx_t, r_total // _NW)
    return out.T.reshape(b, r_total)


# speedup vs baseline: 1.0042x; 1.0042x over previous
import functools

import jax
import jax.numpy as jnp
from jax import lax
from jax.experimental import pallas as pl
from jax.experimental.pallas import tpu as pltpu
from jax.experimental.pallas import tpu_sc as plsc

_NC = 2
_NS = 16
_NW = _NC * _NS

_L = 16
_CH = 4096


@functools.partial(jax.jit, static_argnums=(2,))
def _sc_row_gather(tab, idx, rows_per_w):
    r_total, v = tab.shape
    f_total, b = idx.shape
    d = r_total // f_total
    n_ch = b // _CH
    mesh = plsc.VectorSubcoreMesh(core_axis_name="c", subcore_axis_name="s")

    @functools.partial(
        pl.kernel,
        out_type=jax.ShapeDtypeStruct((r_total, b), jnp.float32),
        mesh=mesh,
        scratch_types=[
            pltpu.VMEM((v,), jnp.float32),
            pltpu.VMEM((b,), jnp.int32),
            pltpu.VMEM((2, _CH), jnp.float32),
            pltpu.SemaphoreType.DMA,
        ],
        compiler_params=pltpu.CompilerParams(needs_layout_passes=False),
    )
    def k(tab_hbm, idx_hbm, out_hbm, row_v, idx_v, out_v, sem_o):
        wid = lax.axis_index("s") * _NC + lax.axis_index("c")
        row0 = wid * rows_per_w

        def drain_out():
            pltpu.make_async_copy(
                out_hbm.at[row0, pl.ds(0, _CH)], out_v.at[0], sem_o
            ).wait()

        @pl.loop(0, rows_per_w, init_carry=jnp.int32(-1))
        def _row(kk, prev_f):
            r = row0 + kk
            f = r // d

            @pl.when(f != prev_f)
            def _():
                pltpu.sync_copy(idx_hbm.at[f], idx_v)

            pltpu.sync_copy(tab_hbm.at[r], row_v)

            for c in range(n_ch):
                @pl.when(kk * n_ch + c >= 2)
                def _():
                    drain_out()

                @plsc.parallel_loop(0, _CH, step=_L, unroll=16)
                def _vec(i):
                    iv = idx_v[pl.ds(c * _CH + i, _L)]
                    out_v[c % 2, pl.ds(i, _L)] = plsc.load_gather(row_v, [iv])

                pltpu.async_copy(
                    out_v.at[c % 2], out_hbm.at[r, pl.ds(c * _CH, _CH)], sem_o
                )
            return f

        drain_out()
        drain_out()

    return k(tab, idx)


def kernel(input, tables):
    f, v, d = tables.shape
    b = input.shape[0]
    r_total = f * d
    assert r_total % _NW == 0 and b % _CH == 0
    tab_rows = tables.transpose(0, 2, 1).reshape(r_total, v)
    idx_t = input.astype(jnp.int32).T
    out = _sc_row_gather(tab_rows, idx_t, r_total // _NW)
    return out.T.reshape(b, r_total)
